# Initial kernel scaffold; baseline (speedup 1.0000x reference)
import jax
import jax.numpy as jnp
from jax.experimental import pallas as pl


def kernel(token_ids, action_actors, action_streets, action_legal_masks, actor_W, street_W, pos_W, mlp_W, mlp_b, ln_g, ln_b):
    raise NotImplementedError("write your pallas kernel here")


# speedup vs baseline: 5.0876x; 5.0876x over previous
import jax
import jax.numpy as jnp
from jax.experimental import pallas as pl
from jax.experimental.pallas import tpu as pltpu

_BB = 8


def _body(tok_ref, a_ref, st_ref, lm_ref,
          actor_W_ref, street_W_ref, pos_ref, mlp_W_ref,
          mlp_b_ref, ln_g_ref, ln_b_ref, out_ref):
    h = jnp.dot(lm_ref[...], mlp_W_ref[...],
                preferred_element_type=jnp.float32)
    h = h + mlp_b_ref[...]
    mu = jnp.mean(h, axis=-1, keepdims=True)
    var = jnp.mean((h - mu) ** 2, axis=-1, keepdims=True)
    h = (h - mu) * jax.lax.rsqrt(var + 1e-5) * ln_g_ref[...] + ln_b_ref[...]
    h = jnp.maximum(h, 0.0)

    a = a_ref[...]
    aW = actor_W_ref[...]
    actor_e = jnp.where(a == 0, aW[0:1, :], aW[1:2, :])

    st = st_ref[...]
    sW = street_W_ref[...]
    street_e = jnp.where(
        st < 2,
        jnp.where(st == 0, sW[0:1, :], sW[1:2, :]),
        jnp.where(st == 2, sW[2:3, :], sW[3:4, :]),
    )

    valid = (tok_ref[...] >= 0).astype(jnp.float32)
    out_ref[...] = (h + actor_e + street_e + pos_ref[...]) * valid


def kernel(token_ids, action_actors, action_streets, action_legal_masks,
           actor_W, street_W, pos_W, mlp_W, mlp_b, ln_g, ln_b):
    B, S = token_ids.shape
    NB = action_legal_masks.shape[-1]
    D = actor_W.shape[-1]
    R = _BB * S
    N = B * S
    grid = (N // R,)

    pos_tiled = jnp.tile(pos_W, (_BB, 1))

    def im_row(i):
        return (i, 0)

    def im_full(i):
        return (0, 0)

    out = pl.pallas_call(
        _body,
        grid=grid,
        in_specs=[
            pl.BlockSpec((R, 1), im_row),
            pl.BlockSpec((R, 1), im_row),
            pl.BlockSpec((R, 1), im_row),
            pl.BlockSpec((R, NB), im_row),
            pl.BlockSpec((2, D), im_full),
            pl.BlockSpec((4, D), im_full),
            pl.BlockSpec((R, D), im_full),
            pl.BlockSpec((NB, D), im_full),
            pl.BlockSpec((1, D), im_full),
            pl.BlockSpec((1, D), im_full),
            pl.BlockSpec((1, D), im_full),
        ],
        out_specs=pl.BlockSpec((R, D), im_row),
        out_shape=jax.ShapeDtypeStruct((N, D), jnp.float32),
    )(token_ids.reshape(N, 1), action_actors.reshape(N, 1),
      action_streets.reshape(N, 1),
      action_legal_masks.astype(jnp.float32).reshape(N, NB),
      actor_W, street_W, pos_tiled, mlp_W,
      mlp_b.reshape(1, D), ln_g.reshape(1, D), ln_b.reshape(1, D))
    return out.reshape(B, S, D)


# speedup vs baseline: 5.7747x; 1.1351x over previous
import jax
import jax.numpy as jnp
from jax.experimental import pallas as pl
from jax.experimental.pallas import tpu as pltpu

_BB = 16


def _body(tok_ref, a_ref, st_ref, lm_ref,
          T_ref, pos_ref, W_ref, b_ref, sb_ref, g_ref, lnb_ref, out_ref):
    D = out_ref.shape[-1]
    inv_d = 1.0 / D

    hs = jnp.dot(lm_ref[...], W_ref[...],
                 preferred_element_type=jnp.float32)
    h = hs[:, :D] + b_ref[...]
    s1 = hs[:, D:D + 1] + sb_ref[...]
    hh = h * h
    s2 = jnp.dot(hh, jnp.ones((D, 1), jnp.float32),
                 preferred_element_type=jnp.float32)
    mu = s1 * inv_d
    var = s2 * inv_d - mu * mu
    p = jax.lax.rsqrt(var + 1e-5)
    q = -(mu * p)
    P = jnp.dot(p, g_ref[...], preferred_element_type=jnp.float32)
    A = jnp.dot(q, g_ref[...], preferred_element_type=jnp.float32)
    r = jnp.maximum(h * P + A + lnb_ref[...], 0.0)

    cf = a_ref[...].astype(jnp.float32) * 4.0 + st_ref[...].astype(jnp.float32)
    cM = jnp.dot(cf, jnp.ones((1, 8), jnp.float32),
                 preferred_element_type=jnp.float32)
    iota8 = jax.lax.broadcasted_iota(jnp.int32, (1, 8), 1).astype(jnp.float32)
    oh = jnp.where(cM == iota8, 1.0, 0.0)
    base = jnp.dot(oh, T_ref[...], preferred_element_type=jnp.float32)

    v = (tok_ref[...] >= 0).astype(jnp.float32)
    vM = jnp.dot(v, jnp.ones((1, D), jnp.float32),
                 preferred_element_type=jnp.float32)

    out_ref[...] = (r + base + pos_ref[...]) * vM


def kernel(token_ids, action_actors, action_streets, action_legal_masks,
           actor_W, street_W, pos_W, mlp_W, mlp_b, ln_g, ln_b):
    B, S = token_ids.shape
    NB = action_legal_masks.shape[-1]
    D = actor_W.shape[-1]
    R = _BB * S
    N = B * S
    grid = (N // R,)

    pos_tiled = jnp.tile(pos_W, (_BB, 1))
    T8 = (actor_W[:, None, :] + street_W[None, :, :]).reshape(8, D)
    W_aug = jnp.concatenate([mlp_W, mlp_W.sum(1, keepdims=True)], axis=1)
    sum_b = mlp_b.sum().reshape(1, 1)

    def im_row(i):
        return (i, 0)

    def im_full(i):
        return (0, 0)

    out = pl.pallas_call(
        _body,
        grid=grid,
        in_specs=[
            pl.BlockSpec((R, 1), im_row),
            pl.BlockSpec((R, 1), im_row),
            pl.BlockSpec((R, 1), im_row),
            pl.BlockSpec((R, NB), im_row),
            pl.BlockSpec((8, D), im_full),
            pl.BlockSpec((R, D), im_full),
            pl.BlockSpec((NB, D + 1), im_full),
            pl.BlockSpec((1, D), im_full),
            pl.BlockSpec((1, 1), im_full),
            pl.BlockSpec((1, D), im_full),
            pl.BlockSpec((1, D), im_full),
        ],
        out_specs=pl.BlockSpec((R, D), im_row),
        out_shape=jax.ShapeDtypeStruct((N, D), jnp.float32),
    )(token_ids.reshape(N, 1), action_actors.reshape(N, 1),
      action_streets.reshape(N, 1),
      action_legal_masks.reshape(N, NB),
      T8, pos_tiled, W_aug,
      mlp_b.reshape(1, D), sum_b, ln_g.reshape(1, D), ln_b.reshape(1, D))
    return out.reshape(B, S, D)


# speedup vs baseline: 6.0075x; 1.0403x over previous
import jax
import jax.numpy as jnp
from jax.experimental import pallas as pl
from jax.experimental.pallas import tpu as pltpu

_BB = 16


def _body(tok_ref, a_ref, st_ref, lm_ref,
          T_ref, pos_ref, W_ref, b_ref, sb_ref, g_ref, lnb_ref, out_ref):
    D = out_ref.shape[-1]
    inv_d = 1.0 / D

    hs = jnp.dot(lm_ref[...], W_ref[...],
                 preferred_element_type=jnp.float32)
    h = hs[:, :D] + b_ref[...]
    s1 = hs[:, D:D + 1] + sb_ref[...]
    hh = h * h
    s2 = jnp.dot(hh, jnp.ones((D, 1), jnp.float32),
                 preferred_element_type=jnp.float32)
    mu = s1 * inv_d
    var = s2 * inv_d - mu * mu
    p = jax.lax.rsqrt(var + 1e-5)
    q = -(mu * p)
    t = h * p + q
    r = jnp.maximum(t * g_ref[...] + lnb_ref[...], 0.0)

    c = a_ref[...] * 4 + st_ref[...]
    iota8 = jax.lax.broadcasted_iota(jnp.int32, (1, 8), 1)
    oh = (c == iota8).astype(jnp.float32)
    base = jnp.dot(oh, T_ref[...], preferred_element_type=jnp.float32)

    v = (tok_ref[...] >= 0).astype(jnp.float32)
    out_ref[...] = (r + base + pos_ref[...]) * v


def kernel(token_ids, action_actors, action_streets, action_legal_masks,
           actor_W, street_W, pos_W, mlp_W, mlp_b, ln_g, ln_b):
    B, S = token_ids.shape
    NB = action_legal_masks.shape[-1]
    D = actor_W.shape[-1]
    R = _BB * S
    N = B * S
    grid = (N // R,)

    pos_tiled = jnp.tile(pos_W, (_BB, 1))
    T8 = (actor_W[:, None, :] + street_W[None, :, :]).reshape(8, D)
    W_aug = jnp.concatenate([mlp_W, mlp_W.sum(1, keepdims=True)], axis=1)
    sum_b = mlp_b.sum().reshape(1, 1)

    def im_row(i):
        return (i, 0)

    def im_full(i):
        return (0, 0)

    out = pl.pallas_call(
        _body,
        grid=grid,
        in_specs=[
            pl.BlockSpec((R, 1), im_row),
            pl.BlockSpec((R, 1), im_row),
            pl.BlockSpec((R, 1), im_row),
            pl.BlockSpec((R, NB), im_row),
            pl.BlockSpec((8, D), im_full),
            pl.BlockSpec((R, D), im_full),
            pl.BlockSpec((NB, D + 1), im_full),
            pl.BlockSpec((1, D), im_full),
            pl.BlockSpec((1, 1), im_full),
            pl.BlockSpec((1, D), im_full),
            pl.BlockSpec((1, D), im_full),
        ],
        out_specs=pl.BlockSpec((R, D), im_row),
        out_shape=jax.ShapeDtypeStruct((N, D), jnp.float32),
    )(token_ids.reshape(N, 1), action_actors.reshape(N, 1),
      action_streets.reshape(N, 1),
      action_legal_masks.reshape(N, NB),
      T8, pos_tiled, W_aug,
      mlp_b.reshape(1, D), sum_b, ln_g.reshape(1, D), ln_b.reshape(1, D))
    return out.reshape(B, S, D)


# speedup vs baseline: 8.9369x; 1.4876x over previous
import jax
import jax.numpy as jnp
from jax.experimental import pallas as pl
from jax.experimental.pallas import tpu as pltpu

_BB = 32


def _body(code_ref, lm_ref, T_ref, pos_ref, W_ref, b_ref, sb_ref,
          g_ref, lnb_ref, out_ref):
    D = out_ref.shape[-1]
    inv_d = 1.0 / D

    lm = lm_ref[...]
    W = W_ref[...]
    h = jnp.dot(lm, W[:, :D],
                preferred_element_type=jnp.float32) + b_ref[...]
    s1 = jnp.dot(lm, W[:, D:D + 1],
                 preferred_element_type=jnp.float32) + sb_ref[...]
    hh = h * h
    s2 = jnp.dot(hh, jnp.ones((D, 1), jnp.float32),
                 preferred_element_type=jnp.float32)
    mu = s1 * inv_d
    var = s2 * inv_d - mu * mu
    p = jax.lax.rsqrt(var + 1e-5)
    q = -(mu * p)
    t = h * p + q
    r = jnp.maximum(t * g_ref[...] + lnb_ref[...], 0.0)

    code = code_ref[...]
    iota16 = jax.lax.broadcasted_iota(jnp.int32, (1, 16), 1)
    oh = (code == iota16).astype(jnp.float32)
    base = jnp.dot(oh, T_ref[...], preferred_element_type=jnp.float32)

    v = (code >= 8).astype(jnp.float32)
    y = r * v + base
    pos = pos_ref[...]
    S = pos.shape[0]
    for k in range(y.shape[0] // S):
        sl = slice(k * S, (k + 1) * S)
        out_ref[sl, :] = y[sl, :] + pos * v[sl, :]


def kernel(token_ids, action_actors, action_streets, action_legal_masks,
           actor_W, street_W, pos_W, mlp_W, mlp_b, ln_g, ln_b):
    B, S = token_ids.shape
    NB = action_legal_masks.shape[-1]
    D = actor_W.shape[-1]
    R = _BB * S
    N = B * S
    grid = (N // R,)

    code = ((token_ids >= 0).astype(jnp.int32) * 8
            + action_actors * 4 + action_streets)
    T8 = (actor_W[:, None, :] + street_W[None, :, :]).reshape(8, D)
    T16 = jnp.concatenate([jnp.zeros((8, D), jnp.float32), T8], axis=0)
    W_aug = jnp.concatenate([mlp_W, mlp_W.sum(1, keepdims=True)], axis=1)
    sum_b = mlp_b.sum().reshape(1, 1)

    def im_row(i):
        return (i, 0)

    def im_full(i):
        return (0, 0)

    out = pl.pallas_call(
        _body,
        grid=grid,
        in_specs=[
            pl.BlockSpec((R, 1), im_row),
            pl.BlockSpec((R, NB), im_row),
            pl.BlockSpec((16, D), im_full),
            pl.BlockSpec((S, D), im_full),
            pl.BlockSpec((NB, D + 1), im_full),
            pl.BlockSpec((1, D), im_full),
            pl.BlockSpec((1, 1), im_full),
            pl.BlockSpec((1, D), im_full),
            pl.BlockSpec((1, D), im_full),
        ],
        out_specs=pl.BlockSpec((R, D), im_row),
        out_shape=jax.ShapeDtypeStruct((N, D), jnp.float32),
    )(code.reshape(N, 1),
      action_legal_masks.reshape(N, NB),
      T16, pos_W, W_aug,
      mlp_b.reshape(1, D), sum_b, ln_g.reshape(1, D), ln_b.reshape(1, D))
    return out.reshape(B, S, D)
